# baseline (speedup 1.0000x reference)
import jax
import jax.numpy as jnp
from jax import lax
from jax.experimental import pallas as pl
from jax.experimental.pallas import tpu as pltpu
from jax.experimental.pallas import tpu_sc as plsc

_PAD = 2
_DW = 64
_DP = 16
_DOUT = _DW + 2 * _DP
_NC = 2
_NS = 16
_NW = _NC * _NS
_TP = 204
_TS = 208
_G0 = 96
_G1 = 112
_SUP = 16


def _gather_concat(we2, wpe_flat, ipair, woff, o1, o2, b):
    ipw = b // _NW
    mesh = plsc.VectorSubcoreMesh(core_axis_name="core", subcore_axis_name="subcore")

    @pl.kernel(
        out_type=jax.ShapeDtypeStruct((b, _TP, _DOUT), jnp.float32),
        mesh=mesh,
        scratch_types=[
            pltpu.VMEM((_SUP * _TS,), jnp.int32),
            pltpu.VMEM((_SUP * _TS,), jnp.int32),
            pltpu.VMEM((_SUP * _TS,), jnp.int32),
            pltpu.VMEM((_SUP * _TS,), jnp.int32),
            pltpu.VMEM((_G0, 2 * _DW), jnp.float32),
            pltpu.VMEM((_G1, 2 * _DW), jnp.float32),
            pltpu.VMEM((_TP, _DOUT), jnp.float32),
            pltpu.VMEM((400 * _DP,), jnp.float32),
            pltpu.SemaphoreType.DMA,
        ],
    )
    def k(we_hbm, wpe_hbm, ip_hbm, wo_hbm, o1_hbm, o2_hbm, o_hbm,
          vip, vwo, vo1, vo2, bp0, bp1, b96, wpv, gsem):
        wid = lax.axis_index("core") * _NS + lax.axis_index("subcore")
        item0 = wid * ipw
        pltpu.sync_copy(wpe_hbm, wpv)

        def emit_block(rb, dst_t0, src, src_t0, nrows):
            offs = vwo[pl.ds(rb + dst_t0, 16)]
            o1v = vo1[pl.ds(rb + dst_t0, 16)]
            o2v = vo2[pl.ds(rb + dst_t0, 16)]
            for l in range(nrows):
                off = offs[l]
                for cc in range(4):
                    b96[dst_t0 + l, pl.ds(cc * 16, 16)] = (
                        src[src_t0 + l, pl.ds(off + cc * 16, 16)])
                b96[dst_t0 + l, pl.ds(_DW, _DP)] = wpv[pl.ds(o1v[l], _DP)]
                b96[dst_t0 + l, pl.ds(_DW + _DP, _DP)] = wpv[pl.ds(o2v[l], _DP)]

        @pl.loop(0, ipw)
        def _(i):
            c = i // _SUP
            r = i - c * _SUP

            @pl.when(r == 0)
            def _stage():
                base = (item0 + c * _SUP) * _TS
                s0 = pltpu.async_copy(ip_hbm.at[pl.ds(base, _SUP * _TS)], vip, gsem)
                s1 = pltpu.async_copy(wo_hbm.at[pl.ds(base, _SUP * _TS)], vwo, gsem)
                s2 = pltpu.async_copy(o1_hbm.at[pl.ds(base, _SUP * _TS)], vo1, gsem)
                s3 = pltpu.async_copy(o2_hbm.at[pl.ds(base, _SUP * _TS)], vo2, gsem)
                s0.wait(); s1.wait(); s2.wait(); s3.wait()

            rb = r * _TS
            g0 = pltpu.async_copy(we_hbm.at[vip.at[pl.ds(rb, _G0)]], bp0, gsem)
            g1 = pltpu.async_copy(we_hbm.at[vip.at[pl.ds(rb + _G0, _G1)]], bp1, gsem)
            g0.wait(); g1.wait()

            @pl.loop(0, _G0 // 16)
            def _rows_lo(blk):
                t0 = blk * 16
                emit_block(rb, t0, bp0, t0, 16)

            @pl.loop(0, 6)
            def _rows_hi(blk):
                t0 = blk * 16
                emit_block(rb, _G0 + t0, bp1, t0, 16)

            emit_block(rb, 192, bp1, 96, 12)

            pltpu.sync_copy(b96, o_hbm.at[item0 + i])

    return k(we2, wpe_flat, ipair, woff, o1, o2)


def kernel(seq_inputs, e1_pos_inputs, e2_pos_inputs, we, wpe):
    b, t = seq_inputs.shape

    si = seq_inputs.astype(jnp.int32)
    e1 = e1_pos_inputs.astype(jnp.int32)
    e2 = e2_pos_inputs.astype(jnp.int32)

    zpad2 = jnp.zeros((b, _PAD), jnp.int32)
    ztail = jnp.zeros((b, _TS - _TP), jnp.int32)
    si_p = jnp.concatenate([zpad2, si, zpad2, ztail], axis=1)

    def edge_pad(x):
        head = jnp.repeat(x[:, :1], _PAD, axis=1)
        tail = jnp.repeat(x[:, -1:], _PAD, axis=1)
        return jnp.concatenate([head, x, tail, ztail], axis=1)

    e1_p = edge_pad(e1)
    e2_p = edge_pad(e2)

    ipair = (si_p >> 1).reshape(-1)
    woff = ((si_p & 1) << 6).reshape(-1)
    o1 = (e1_p * _DP).reshape(-1)
    o2 = (e2_p * _DP).reshape(-1)

    out = _gather_concat(we.reshape(we.shape[0] // 2, 2 * _DW),
                         wpe.reshape(-1), ipair, woff, o1, o2, b)
    return out


# speedup vs baseline: 1.0041x; 1.0041x over previous
import jax
import jax.numpy as jnp
from jax import lax
from jax.experimental import pallas as pl
from jax.experimental.pallas import tpu as pltpu
from jax.experimental.pallas import tpu_sc as plsc

_PAD = 2
_DW = 64
_DP = 16
_DOUT = _DW + 2 * _DP
_NC = 2
_NS = 16
_NW = _NC * _NS
_TP = 204
_TS = 208
_G0 = 96
_G1 = 112
_SUP = 16


def _lane(v, l):
    return jnp.take_along_axis(v, jnp.full((16,), l, jnp.int32), axis=0)


def _gather_concat(we2, wpe_flat, ipair, wmix, b):
    ipw = b // _NW
    mesh = plsc.VectorSubcoreMesh(core_axis_name="core", subcore_axis_name="subcore")

    @pl.kernel(
        out_type=jax.ShapeDtypeStruct((b, _TP, _DOUT), jnp.float32),
        mesh=mesh,
        compiler_params=pltpu.CompilerParams(needs_layout_passes=False),
        scratch_types=[
            pltpu.VMEM((_SUP * _TS,), jnp.int32),
            pltpu.VMEM((_SUP * _TS,), jnp.int32),
            pltpu.VMEM((_G0, 2 * _DW), jnp.float32),
            pltpu.VMEM((_G1, 2 * _DW), jnp.float32),
            pltpu.VMEM((_TP, _DOUT), jnp.float32),
            pltpu.VMEM((400 * _DP,), jnp.float32),
            pltpu.SemaphoreType.DMA,
        ],
    )
    def k(we_hbm, wpe_hbm, ip_hbm, wm_hbm, o_hbm,
          vip, vmix, bp0, bp1, b96, wpv, gsem):
        wid = lax.axis_index("core") * _NS + lax.axis_index("subcore")
        item0 = wid * ipw
        pltpu.sync_copy(wpe_hbm, wpv)
        iota = lax.iota(jnp.int32, 16)

        def emit_block(rb, dst_t0, src, src_t0, nrows):
            wv = vmix[pl.ds(rb + dst_t0, 16)]
            offv = jnp.right_shift(wv, 25) & 64
            o1v = wv & 0xFFFF
            o2v = jnp.right_shift(wv, 16) & 0x7FFF
            for l in range(nrows):
                rowc = jnp.full((16,), src_t0 + l, jnp.int32)
                colb = _lane(offv, l) + iota
                for cc in range(4):
                    b96[dst_t0 + l, pl.ds(cc * 16, 16)] = plsc.load_gather(
                        src, [rowc, colb + cc * 16])
                b96[dst_t0 + l, pl.ds(_DW, _DP)] = plsc.load_gather(
                    wpv, [_lane(o1v, l) + iota])
                b96[dst_t0 + l, pl.ds(_DW + _DP, _DP)] = plsc.load_gather(
                    wpv, [_lane(o2v, l) + iota])

        @pl.loop(0, ipw)
        def _(i):
            c = i // _SUP
            r = i - c * _SUP

            @pl.when(r == 0)
            def _stage():
                base = (item0 + c * _SUP) * _TS
                s0 = pltpu.async_copy(ip_hbm.at[pl.ds(base, _SUP * _TS)], vip, gsem)
                s1 = pltpu.async_copy(wm_hbm.at[pl.ds(base, _SUP * _TS)], vmix, gsem)
                s0.wait(); s1.wait()

            rb = r * _TS
            g0 = pltpu.async_copy(we_hbm.at[vip.at[pl.ds(rb, _G0)]], bp0, gsem)
            g1 = pltpu.async_copy(we_hbm.at[vip.at[pl.ds(rb + _G0, _G1)]], bp1, gsem)
            g0.wait(); g1.wait()

            @pl.loop(0, _G0 // 16)
            def _rows_lo(blk):
                t0 = blk * 16
                emit_block(rb, t0, bp0, t0, 16)

            @pl.loop(0, 6)
            def _rows_hi(blk):
                t0 = blk * 16
                emit_block(rb, _G0 + t0, bp1, t0, 16)

            emit_block(rb, 192, bp1, 96, 12)

            pltpu.sync_copy(b96, o_hbm.at[item0 + i])

    return k(we2, wpe_flat, ipair, wmix)


def kernel(seq_inputs, e1_pos_inputs, e2_pos_inputs, we, wpe):
    b, t = seq_inputs.shape

    si = seq_inputs.astype(jnp.int32)
    e1 = e1_pos_inputs.astype(jnp.int32)
    e2 = e2_pos_inputs.astype(jnp.int32)

    zpad2 = jnp.zeros((b, _PAD), jnp.int32)
    ztail = jnp.zeros((b, _TS - _TP), jnp.int32)
    si_p = jnp.concatenate([zpad2, si, zpad2, ztail], axis=1)

    def edge_pad(x):
        head = jnp.repeat(x[:, :1], _PAD, axis=1)
        tail = jnp.repeat(x[:, -1:], _PAD, axis=1)
        return jnp.concatenate([head, x, tail, ztail], axis=1)

    e1_p = edge_pad(e1)
    e2_p = edge_pad(e2)

    ipair = (si_p >> 1).reshape(-1)
    wmix = (((si_p & 1) << 31) | ((e2_p * _DP) << 16) | (e1_p * _DP)).reshape(-1)

    out = _gather_concat(we.reshape(we.shape[0] // 2, 2 * _DW),
                         wpe.reshape(-1), ipair, wmix, b)
    return out
